# baseline (speedup 1.0000x reference)
import jax
import jax.numpy as jnp
from jax import lax
from jax.experimental import pallas as pl
from jax.experimental.pallas import tpu as pltpu

_ROWS = 1_000_000
_DIM = 16
_NCHUNK = 8
_CHUNK = 124928
_TAIL_OFF = _NCHUNK * _CHUNK
_TAIL = _ROWS - _TAIL_OFF


def _copy_body(u_ref, i_ref, ou_ref, oi_ref, sems):
    copies = []
    for k in range(_NCHUNK):
        sl = pl.ds(k * _CHUNK, _CHUNK)
        copies.append(
            pltpu.make_async_copy(u_ref.at[:, sl], ou_ref.at[:, sl], sems.at[2 * k])
        )
        copies.append(
            pltpu.make_async_copy(i_ref.at[:, sl], oi_ref.at[:, sl], sems.at[2 * k + 1])
        )
    tl = pl.ds(_TAIL_OFF, _TAIL)
    copies.append(
        pltpu.make_async_copy(u_ref.at[:, tl], ou_ref.at[:, tl], sems.at[2 * _NCHUNK])
    )
    copies.append(
        pltpu.make_async_copy(i_ref.at[:, tl], oi_ref.at[:, tl], sems.at[2 * _NCHUNK + 1])
    )
    for c in copies:
        c.start()
    for c in copies:
        c.wait()


def kernel(user_table, item_table):
    out = pl.pallas_call(
        _copy_body,
        in_specs=[
            pl.BlockSpec(memory_space=pl.ANY),
            pl.BlockSpec(memory_space=pl.ANY),
        ],
        out_specs=[
            pl.BlockSpec(memory_space=pl.ANY),
            pl.BlockSpec(memory_space=pl.ANY),
        ],
        out_shape=[
            jax.ShapeDtypeStruct((_DIM, _ROWS), user_table.dtype),
            jax.ShapeDtypeStruct((_DIM, _ROWS), item_table.dtype),
        ],
        scratch_shapes=[pltpu.SemaphoreType.DMA((2 * _NCHUNK + 2,))],
    )(user_table.T, item_table.T)
    return (out[0].T, out[1].T)


# speedup vs baseline: 49.0178x; 49.0178x over previous
import jax
import jax.numpy as jnp
from jax import lax
from jax.experimental import pallas as pl
from jax.experimental.pallas import tpu as pltpu

_ROWS = 1_000_000
_DIM = 16
_BLK = 65536
_GRID = (_ROWS + _BLK - 1) // _BLK


def _copy_body(u_ref, i_ref, ou_ref, oi_ref):
    ou_ref[...] = u_ref[...]
    oi_ref[...] = i_ref[...]


def kernel(user_table, item_table):
    spec = pl.BlockSpec((_DIM, _BLK), lambda k: (0, k))
    out = pl.pallas_call(
        _copy_body,
        grid=(_GRID,),
        in_specs=[spec, spec],
        out_specs=[spec, spec],
        out_shape=[
            jax.ShapeDtypeStruct((_DIM, _ROWS), user_table.dtype),
            jax.ShapeDtypeStruct((_DIM, _ROWS), item_table.dtype),
        ],
    )(user_table.T, item_table.T)
    return (out[0].T, out[1].T)


# speedup vs baseline: 49.1306x; 1.0023x over previous
import jax
import jax.numpy as jnp
from jax import lax
from jax.experimental import pallas as pl
from jax.experimental.pallas import tpu as pltpu

_ROWS = 1_000_000
_DIM = 16
_BLK = 98304
_GRID = (_ROWS + _BLK - 1) // _BLK


def _copy_body(u_ref, i_ref, ou_ref, oi_ref):
    ou_ref[...] = u_ref[...]
    oi_ref[...] = i_ref[...]


def kernel(user_table, item_table):
    spec = pl.BlockSpec((_DIM, _BLK), lambda k: (0, k))
    out = pl.pallas_call(
        _copy_body,
        grid=(_GRID,),
        in_specs=[spec, spec],
        out_specs=[spec, spec],
        out_shape=[
            jax.ShapeDtypeStruct((_DIM, _ROWS), user_table.dtype),
            jax.ShapeDtypeStruct((_DIM, _ROWS), item_table.dtype),
        ],
    )(user_table.T, item_table.T)
    return (out[0].T, out[1].T)
